# baseline (speedup 1.0000x reference)
import functools
import jax
import jax.numpy as jnp
import numpy as np
from jax import lax
from jax.experimental import pallas as pl
from jax.experimental.pallas import tpu as pltpu
from jax.experimental.pallas import tpu_sc as plsc

N = 10000
E = 320000
D = 128
H = 8
DH = 16
CLAMP = 5.0

NC = 2
NS = 16
NW = NC * NS
EPW = E // NW


def _qkv_body(x_ref, w_ref, q_ref, kv_ref):
    qkv = jnp.dot(x_ref[...], w_ref[...], preferred_element_type=jnp.float32)
    q_ref[...] = qkv[:, :D]
    kv_ref[...] = qkv[:, D:]


def _qkv(x, wqkv):
    return pl.pallas_call(
        _qkv_body,
        out_shape=(jax.ShapeDtypeStruct((N, D), jnp.float32),
                   jax.ShapeDtypeStruct((N, 2 * D), jnp.float32)),
    )(x, wqkv)


NROW = E // 128
RPT = NROW // NW


def _gather_body(qh, kvh, dst2, src2, qd_o, kvs_o,
                 idx_d, idx_s, qb0, qb1, kvb0, kvb1,
                 gsem0, gsem1, wsem0, wsem1):
    wid = lax.axis_index("c") * NS + lax.axis_index("s")
    start = wid * RPT
    pltpu.sync_copy(dst2.at[pl.ds(start, RPT)], idx_d)
    pltpu.sync_copy(src2.at[pl.ds(start, RPT)], idx_s)

    qb = (qb0, qb1)
    kvb = (kvb0, kvb1)
    gsem = (gsem0, gsem1)
    wsem = (wsem0, wsem1)
    cps = {}
    for i in range(RPT):
        sl = i & 1
        nsl = sl ^ 1
        if i == 0:
            cps['gq', 0] = pltpu.async_copy(qh.at[idx_d.at[0]], qb[0],
                                            gsem[0])
            cps['gkv', 0] = pltpu.async_copy(kvh.at[idx_s.at[0]], kvb[0],
                                             gsem[0])
        if i + 1 < RPT:
            if i >= 1:
                cps['wq', i - 1].wait()
                cps['wkv', i - 1].wait()
            cps['gq', i + 1] = pltpu.async_copy(qh.at[idx_d.at[i + 1]],
                                                qb[nsl], gsem[nsl])
            cps['gkv', i + 1] = pltpu.async_copy(kvh.at[idx_s.at[i + 1]],
                                                 kvb[nsl], gsem[nsl])
        cps['gq', i].wait()
        cps['gkv', i].wait()
        ebase = (start + i) * 128
        cps['wq', i] = pltpu.async_copy(qb[sl], qd_o.at[pl.ds(ebase, 128)],
                                        wsem[sl])
        cps['wkv', i] = pltpu.async_copy(kvb[sl],
                                         kvs_o.at[pl.ds(ebase, 128)],
                                         wsem[sl])
    for i in (RPT - 2, RPT - 1):
        cps['wq', i].wait()
        cps['wkv', i].wait()

    @pl.when(wid < NROW - NW * RPT)
    def _():
        row = NW * RPT + wid
        pltpu.sync_copy(dst2.at[pl.ds(row, 1)], idx_d.at[pl.ds(0, 1)])
        pltpu.sync_copy(src2.at[pl.ds(row, 1)], idx_s.at[pl.ds(0, 1)])
        pltpu.async_copy(qh.at[idx_d.at[0]], qb[0], gsem[0]).wait()
        pltpu.async_copy(kvh.at[idx_s.at[0]], kvb[0], gsem[0]).wait()
        pltpu.sync_copy(qb[0], qd_o.at[pl.ds(row * 128, 128)])
        pltpu.sync_copy(kvb[0], kvs_o.at[pl.ds(row * 128, 128)])


def _sc_gather(qh, kvh, dst2, src2):
    mesh = plsc.VectorSubcoreMesh(core_axis_name="c", subcore_axis_name="s")
    f = pl.kernel(
        _gather_body,
        out_type=(
            jax.ShapeDtypeStruct((E, D), jnp.float32),
            jax.ShapeDtypeStruct((E, 2 * D), jnp.float32),
        ),
        mesh=mesh,
        scratch_types=[
            pltpu.VMEM((RPT, 128), jnp.int32),
            pltpu.VMEM((RPT, 128), jnp.int32),
            pltpu.VMEM((128, D), jnp.float32),
            pltpu.VMEM((128, D), jnp.float32),
            pltpu.VMEM((128, 2 * D), jnp.float32),
            pltpu.VMEM((128, 2 * D), jnp.float32),
            pltpu.SemaphoreType.DMA,
            pltpu.SemaphoreType.DMA,
            pltpu.SemaphoreType.DMA,
            pltpu.SemaphoreType.DMA,
        ],
        compiler_params=pltpu.CompilerParams(use_tc_tiling_on_sc=False),
    )
    return f(qh, kvh, dst2, src2)


EB = 512


def _edge_body(fc_ref, qd_ref, kv_ref, wc_ref, bc_ref, w2_ref,
               a2_ref, s_ref, conn2_ref, w_ref, contrib_ref):
    fc = fc_ref[...]
    eh = jnp.dot(fc, wc_ref[...], preferred_element_type=jnp.float32)
    eh = eh + bc_ref[...]
    ew = eh[:, :D]
    ebias = eh[:, D:]
    kv = kv_ref[...]
    conn = (qd_ref[...] + kv[:, :D]) * ew
    conn = jnp.sign(conn) * jnp.sqrt(jnp.abs(conn))
    conn = jnp.maximum(conn + ebias, 0.0)
    conn2_ref[...] = jnp.dot(conn, w2_ref[...],
                             preferred_element_type=jnp.float32)
    conn2 = jnp.maximum(conn2_ref[...] + fc, 0.0)
    conn2_ref[...] = conn2
    score = jnp.dot(conn, a2_ref[...], preferred_element_type=jnp.float32)
    w = jnp.exp(jnp.clip(score, -CLAMP, CLAMP))
    w_ref[...] = w
    wide = jnp.dot(w, s_ref[...], preferred_element_type=jnp.float32)
    contrib_ref[...] = wide * (kv[:, D:] + conn2)


def _tc_edge(fc, qd, kvs, wc, bc, w2, a2, sel):
    grid = (E // EB,)
    eb_spec = pl.BlockSpec((EB, D), lambda i: (i, 0))
    kv_spec = pl.BlockSpec((EB, 2 * D), lambda i: (i, 0))
    w_spec = pl.BlockSpec((EB, H), lambda i: (i, 0))
    full = lambda shape: pl.BlockSpec(shape, lambda i: tuple(0 for _ in shape))
    return pl.pallas_call(
        _edge_body,
        grid=grid,
        in_specs=[eb_spec, eb_spec, kv_spec,
                  full((D, 2 * D)), full((1, 2 * D)), full((D, D)),
                  full((D, H)), full((H, D))],
        out_specs=[eb_spec, w_spec, eb_spec],
        out_shape=(
            jax.ShapeDtypeStruct((E, D), jnp.float32),
            jax.ShapeDtypeStruct((E, H), jnp.float32),
            jax.ShapeDtypeStruct((E, D), jnp.float32),
        ),
        compiler_params=pltpu.CompilerParams(
            dimension_semantics=("arbitrary",),
        ),
    )(fc, qd, kvs, wc, bc, w2, a2, sel)


def _scatter_body(contrib, w, dst2, zacc, zs, acc_o, s_o, acc_sh, s_sh,
                  idx_d, rb0, rb1, wb0, wb1, lsem0, lsem1, asem0, asem1):
    c = lax.axis_index("c")
    s = lax.axis_index("s")
    wid = c * NS + s
    start = wid * RPT

    @pl.when(s == 0)
    def _():
        pltpu.sync_copy(zacc, acc_sh)
        pltpu.sync_copy(zs, s_sh)
    plsc.subcore_barrier()

    pltpu.sync_copy(dst2.at[pl.ds(start, RPT)], idx_d)

    rb = (rb0, rb1)
    wb = (wb0, wb1)
    lsem = (lsem0, lsem1)
    asem = (asem0, asem1)
    cps = {}
    for i in range(RPT):
        sl = i & 1
        nsl = sl ^ 1
        if i == 0:
            cps['lr', 0] = pltpu.async_copy(
                contrib.at[pl.ds(start * 128, 128)], rb[0], lsem[0])
            cps['lw', 0] = pltpu.async_copy(
                w.at[pl.ds(start * 128, 128)], wb[0], lsem[0])
        if i + 1 < RPT:
            if i >= 1:
                cps['ar', i - 1].wait()
                cps['aw', i - 1].wait()
            ebase = (start + i + 1) * 128
            cps['lr', i + 1] = pltpu.async_copy(
                contrib.at[pl.ds(ebase, 128)], rb[nsl], lsem[nsl])
            cps['lw', i + 1] = pltpu.async_copy(
                w.at[pl.ds(ebase, 128)], wb[nsl], lsem[nsl])
        cps['lr', i].wait()
        cps['lw', i].wait()
        cps['ar', i] = pltpu.async_copy(rb[sl], acc_sh.at[idx_d.at[i]],
                                        asem[sl], add=True)
        cps['aw', i] = pltpu.async_copy(wb[sl], s_sh.at[idx_d.at[i]],
                                        asem[sl], add=True)
    for i in (RPT - 2, RPT - 1):
        cps['ar', i].wait()
        cps['aw', i].wait()

    @pl.when(wid < NROW - NW * RPT)
    def _():
        row = NW * RPT + wid
        pltpu.sync_copy(dst2.at[pl.ds(row, 1)], idx_d.at[pl.ds(0, 1)])
        pltpu.sync_copy(contrib.at[pl.ds(row * 128, 128)], rb[0])
        pltpu.sync_copy(w.at[pl.ds(row * 128, 128)], wb[0])
        pltpu.sync_copy(rb[0], acc_sh.at[idx_d.at[0]], add=True)
        pltpu.sync_copy(wb[0], s_sh.at[idx_d.at[0]], add=True)

    plsc.subcore_barrier()

    @pl.when(s == 0)
    def _():
        pltpu.sync_copy(acc_sh, acc_o.at[c])
        pltpu.sync_copy(s_sh, s_o.at[c])


def _sc_scatter(contrib, w, dst2, zacc, zs):
    mesh = plsc.VectorSubcoreMesh(core_axis_name="c", subcore_axis_name="s")
    f = pl.kernel(
        _scatter_body,
        out_type=(
            jax.ShapeDtypeStruct((NC, N, D), jnp.float32),
            jax.ShapeDtypeStruct((NC, N, H), jnp.float32),
        ),
        mesh=mesh,
        scratch_types=[
            pltpu.VMEM_SHARED((N, D), jnp.float32),
            pltpu.VMEM_SHARED((N, H), jnp.float32),
            pltpu.VMEM((RPT, 128), jnp.int32),
            pltpu.VMEM((128, D), jnp.float32),
            pltpu.VMEM((128, D), jnp.float32),
            pltpu.VMEM((128, H), jnp.float32),
            pltpu.VMEM((128, H), jnp.float32),
            pltpu.SemaphoreType.DMA,
            pltpu.SemaphoreType.DMA,
            pltpu.SemaphoreType.DMA,
            pltpu.SemaphoreType.DMA,
        ],
        compiler_params=pltpu.CompilerParams(use_tc_tiling_on_sc=False),
    )
    return f(contrib, w, dst2, zacc, zs)


def _node_body(acc_ref, s_ref, x_ref, sd_ref, sel_ref, c0_ref, c1_ref,
               f1w_ref, f1b_ref, f2w_ref, f2b_ref, g1_ref, b1_ref,
               g2_ref, b2_ref, out_ref):
    acc = acc_ref[0] + acc_ref[1]
    sden = s_ref[0] + s_ref[1]
    swide = jnp.dot(sden, sel_ref[...], preferred_element_type=jnp.float32)
    nh = acc / (swide + 1e-16)
    x = x_ref[...]
    nh = nh * (c0_ref[...] + sd_ref[...] * c1_ref[...])
    nh = nh + x
    h_res = nh

    m = jnp.mean(nh, axis=0, keepdims=True)
    v = jnp.mean((nh - m) * (nh - m), axis=0, keepdims=True)
    nh = (nh - m) / jnp.sqrt(v + 1e-5) * g1_ref[...] + b1_ref[...]

    nh = jnp.dot(nh, f1w_ref[...], preferred_element_type=jnp.float32)
    nh = jnp.maximum(nh + f1b_ref[...], 0.0)
    nh = jnp.dot(nh, f2w_ref[...], preferred_element_type=jnp.float32)
    nh = nh + f2b_ref[...] + h_res

    m2 = jnp.mean(nh, axis=0, keepdims=True)
    v2 = jnp.mean((nh - m2) * (nh - m2), axis=0, keepdims=True)
    out_ref[...] = (nh - m2) / jnp.sqrt(v2 + 1e-5) * g2_ref[...] + b2_ref[...]


def _tc_node(acc, s, x, sqrt_deg, sel, c0, c1, f1w, f1b, f2w, f2b,
             g1, b1, g2, b2):
    return pl.pallas_call(
        _node_body,
        out_shape=jax.ShapeDtypeStruct((N, D), jnp.float32),
        compiler_params=pltpu.CompilerParams(
            vmem_limit_bytes=120 * 1024 * 1024,
        ),
    )(acc, s, x, sqrt_deg, sel, c0, c1, f1w, f1b, f2w, f2b, g1, b1, g2, b2)


def kernel(x, full_index, full_conn, sqrt_deg, qkv_w, qkv_b, conn_w, conn_b,
           Aw, Bw, deg_coef, ffn1_w, ffn1_b, ffn2_w, ffn2_b, bn1_g, bn1_b,
           bn2_g, bn2_b):
    wqkv = qkv_w.T
    wc = conn_w.T
    bc = conn_b.reshape(1, 2 * D)
    hh = jnp.arange(D) // DH
    blk = (hh[:, None] == hh[None, :]).astype(jnp.float32)
    w2 = blk * jnp.tile(Bw.transpose(1, 0, 2).reshape(H * DH, DH), (1, H))
    a2 = (jnp.arange(H)[None, :] == hh[:, None]).astype(jnp.float32) * \
        Aw[:, :, 0].T.reshape(D, 1)
    sel = (hh[None, :] == jnp.arange(H)[:, None]).astype(jnp.float32)
    c0 = deg_coef[0, :, 0].reshape(1, D)
    c1 = deg_coef[0, :, 1].reshape(1, D)

    dst2 = full_index[0].reshape(NROW, 128)
    src2 = full_index[1].reshape(NROW, 128)
    zacc = jnp.zeros((N, D), jnp.float32)
    zs = jnp.zeros((N, H), jnp.float32)

    qh, kvh = _qkv(x, wqkv)

    qd, kvs = _sc_gather(qh, kvh, dst2, src2)
    conn2, w, contrib = _tc_edge(full_conn, qd, kvs, wc, bc, w2, a2, sel)
    acc, s = _sc_scatter(contrib, w, dst2, zacc, zs)
    nh = _tc_node(acc, s, x, sqrt_deg, sel, c0, c1,
                  ffn1_w.T, ffn1_b.reshape(1, 2 * D),
                  ffn2_w.T, ffn2_b.reshape(1, D),
                  bn1_g.reshape(1, D), bn1_b.reshape(1, D),
                  bn2_g.reshape(1, D), bn2_b.reshape(1, D))
    return nh, conn2


# speedup vs baseline: 1.6266x; 1.6266x over previous
import functools
import jax
import jax.numpy as jnp
import numpy as np
from jax import lax
from jax.experimental import pallas as pl
from jax.experimental.pallas import tpu as pltpu
from jax.experimental.pallas import tpu_sc as plsc

N = 10000
E = 320000
D = 128
H = 8
DH = 16
CLAMP = 5.0

NC = 2
NS = 16
NW = NC * NS
EPW = E // NW


def _qkv_body(x_ref, w_ref, q_ref, k_ref, v_ref):
    qkv = jnp.dot(x_ref[...], w_ref[...], preferred_element_type=jnp.float32)
    q_ref[...] = qkv[:, :D]
    k_ref[...] = qkv[:, D:2 * D]
    v_ref[...] = qkv[:, 2 * D:]


def _qkv(x, wqkv):
    return pl.pallas_call(
        _qkv_body,
        out_shape=(jax.ShapeDtypeStruct((N, D), jnp.float32),
                   jax.ShapeDtypeStruct((N, D), jnp.float32),
                   jax.ShapeDtypeStruct((N, D), jnp.float32)),
    )(x, wqkv)


NROW = E // 128
SLABS = ((0, 26, 8), (840, 26, 8), (1680, 25, 20))
GK = 6


def _make_gather_body(row0, rpt, nx, gk):
    def body(qh, kh, vh, dst2, src2, qd_o, ks_o, vs_o,
             idx_d, idx_s, rows_v, sem):
        wid = lax.axis_index("c") * NS + lax.axis_index("s")
        start = row0 + wid * rpt
        lstart = wid * rpt
        pltpu.sync_copy(dst2.at[pl.ds(start, rpt)], idx_d)
        pltpu.sync_copy(src2.at[pl.ds(start, rpt)], idx_s)

        def chunk_at(r0, k):
            for table, idxref, out in ((qh, idx_d, qd_o), (kh, idx_s, ks_o),
                                       (vh, idx_s, vs_o)):
                cps = [pltpu.async_copy(table.at[idxref.at[r0 + j]],
                                        rows_v.at[pl.ds(j * 128, 128)], sem)
                       for j in range(k)]
                for cp in cps:
                    cp.wait()
                pltpu.sync_copy(rows_v.at[pl.ds(0, k * 128)],
                                out.at[pl.ds((lstart + r0) * 128, k * 128)])

        def chunk(i, _):
            chunk_at(i * gk, gk)
            return 0

        lax.fori_loop(0, rpt // gk, chunk, 0)
        if rpt % gk:
            chunk_at((rpt // gk) * gk, rpt % gk)

        if nx:
            @pl.when(wid < nx)
            def _():
                row = row0 + NW * rpt + wid
                lrow = NW * rpt + wid
                pltpu.sync_copy(dst2.at[pl.ds(row, 1)], idx_d.at[pl.ds(0, 1)])
                pltpu.sync_copy(src2.at[pl.ds(row, 1)], idx_s.at[pl.ds(0, 1)])
                for table, idxref, out in ((qh, idx_d, qd_o),
                                           (kh, idx_s, ks_o),
                                           (vh, idx_s, vs_o)):
                    pltpu.async_copy(table.at[idxref.at[0]],
                                     rows_v.at[pl.ds(0, 128)], sem).wait()
                    pltpu.sync_copy(rows_v.at[pl.ds(0, 128)],
                                    out.at[pl.ds(lrow * 128, 128)])
    return body


def _sc_gather(qh, kh, vh, dst2, src2, row0, rpt, nx, gk=GK):
    nedge = (NW * rpt + nx) * 128
    mesh = plsc.VectorSubcoreMesh(core_axis_name="c", subcore_axis_name="s")
    f = pl.kernel(
        _make_gather_body(row0, rpt, nx, gk),
        out_type=(
            jax.ShapeDtypeStruct((nedge, D), jnp.float32),
            jax.ShapeDtypeStruct((nedge, D), jnp.float32),
            jax.ShapeDtypeStruct((nedge, D), jnp.float32),
        ),
        mesh=mesh,
        scratch_types=[
            pltpu.VMEM((rpt, 128), jnp.int32),
            pltpu.VMEM((rpt, 128), jnp.int32),
            pltpu.VMEM((gk * 128, D), jnp.float32),
            pltpu.SemaphoreType.DMA,
        ],
        compiler_params=pltpu.CompilerParams(use_tc_tiling_on_sc=False),
    )
    return f(qh, kh, vh, dst2, src2)


EB = 1280


def _edge_body(fc_ref, qk_a_ref, qk_b_ref, v_ref, wc_ref, bc_ref, w2_ref,
               a2_ref, s_ref, conn2_ref, w_ref, contrib_ref):
    fc = fc_ref[...]
    eh = jnp.dot(fc, wc_ref[...], preferred_element_type=jnp.float32)
    eh = eh + bc_ref[...]
    ew = eh[:, :D]
    ebias = eh[:, D:]
    conn = (qk_a_ref[...] + qk_b_ref[...]) * ew
    conn = jnp.sign(conn) * jnp.sqrt(jnp.abs(conn))
    conn = jnp.maximum(conn + ebias, 0.0)
    conn2_ref[...] = jnp.dot(conn, w2_ref[...],
                             preferred_element_type=jnp.float32)
    conn2 = jnp.maximum(conn2_ref[...] + fc, 0.0)
    conn2_ref[...] = conn2
    score = jnp.dot(conn, a2_ref[...], preferred_element_type=jnp.float32)
    w = jnp.exp(jnp.clip(score, -CLAMP, CLAMP))
    w_ref[...] = w
    wide = jnp.dot(w, s_ref[...], preferred_element_type=jnp.float32)
    contrib_ref[...] = wide * (v_ref[...] + conn2)


def _tc_edge(fc, qd, ks, vs, wc, bc, w2, a2, sel, blk0, donor=None):
    nblk = qd.shape[0] // EB
    eb_g = pl.BlockSpec((EB, D), lambda i: (i + blk0, 0))
    eb_l = pl.BlockSpec((EB, D), lambda i: (i, 0))
    w_spec = pl.BlockSpec((EB, H), lambda i: (i, 0))
    full = lambda shape: pl.BlockSpec(shape, lambda i: tuple(0 for _ in shape))
    in_specs = [eb_g, eb_l, eb_l, eb_l,
                full((D, 2 * D)), full((1, 2 * D)), full((D, D)),
                full((D, H)), full((H, D))]
    args = [fc, qd, ks, vs, wc, bc, w2, a2, sel]
    kwargs = {}
    body = _edge_body
    if donor is not None:
        in_specs.append(pl.BlockSpec(memory_space=pl.ANY))
        args.append(donor)
        kwargs['input_output_aliases'] = {9: 0}

        def body(fc_ref, qa, qb, v, wc_r, bc_r, w2_r, a2_r, s_r, donor_ref,
                 conn2_ref, w_ref, contrib_ref):
            del donor_ref
            _edge_body(fc_ref, qa, qb, v, wc_r, bc_r, w2_r, a2_r, s_r,
                       conn2_ref, w_ref, contrib_ref)

    return pl.pallas_call(
        body,
        grid=(nblk,),
        in_specs=in_specs,
        out_specs=[eb_g, w_spec, eb_l],
        out_shape=(
            jax.ShapeDtypeStruct((E, D), jnp.float32),
            jax.ShapeDtypeStruct((qd.shape[0], H), jnp.float32),
            jax.ShapeDtypeStruct((qd.shape[0], D), jnp.float32),
        ),
        compiler_params=pltpu.CompilerParams(
            dimension_semantics=("arbitrary",),
        ),
        **kwargs,
    )(*args)


SK = 2


def _make_scatter_body(row0, rpt, nx):
    def body(contrib, w, dst2, zacc, zs, acc_o, s_o, acc_sh, s_sh,
             idx_d, rows_v, w_v):
        c = lax.axis_index("c")
        s = lax.axis_index("s")
        wid = c * NS + s
        start = row0 + wid * rpt
        lstart = wid * rpt

        @pl.when(s == 0)
        def _():
            pltpu.sync_copy(zacc, acc_sh)
            pltpu.sync_copy(zs, s_sh)
        plsc.subcore_barrier()

        pltpu.sync_copy(dst2.at[pl.ds(start, rpt)], idx_d)

        def chunk_at(r0, k):
            pltpu.sync_copy(contrib.at[pl.ds((lstart + r0) * 128, k * 128)],
                            rows_v.at[pl.ds(0, k * 128)])
            pltpu.sync_copy(w.at[pl.ds((lstart + r0) * 128, k * 128)],
                            w_v.at[pl.ds(0, k * 128)])
            for j in range(k):
                pltpu.sync_copy(rows_v.at[pl.ds(j * 128, 128)],
                                acc_sh.at[idx_d.at[r0 + j]], add=True)
                pltpu.sync_copy(w_v.at[pl.ds(j * 128, 128)],
                                s_sh.at[idx_d.at[r0 + j]], add=True)

        def chunk(i, _):
            chunk_at(i * SK, SK)
            return 0

        lax.fori_loop(0, rpt // SK, chunk, 0)
        if rpt % SK:
            chunk_at((rpt // SK) * SK, rpt % SK)

        if nx:
            @pl.when(wid < nx)
            def _():
                row = row0 + NW * rpt + wid
                lrow = NW * rpt + wid
                pltpu.sync_copy(dst2.at[pl.ds(row, 1)], idx_d.at[pl.ds(0, 1)])
                pltpu.sync_copy(contrib.at[pl.ds(lrow * 128, 128)],
                                rows_v.at[pl.ds(0, 128)])
                pltpu.sync_copy(w.at[pl.ds(lrow * 128, 128)],
                                w_v.at[pl.ds(0, 128)])
                pltpu.sync_copy(rows_v.at[pl.ds(0, 128)],
                                acc_sh.at[idx_d.at[0]], add=True)
                pltpu.sync_copy(w_v.at[pl.ds(0, 128)], s_sh.at[idx_d.at[0]],
                                add=True)

        plsc.subcore_barrier()

        @pl.when(s == 0)
        def _():
            pltpu.sync_copy(acc_sh, acc_o.at[c])
            pltpu.sync_copy(s_sh, s_o.at[c])
    return body


def _sc_scatter(contrib, w, dst2, zacc, zs, row0, rpt, nx):
    mesh = plsc.VectorSubcoreMesh(core_axis_name="c", subcore_axis_name="s")
    f = pl.kernel(
        _make_scatter_body(row0, rpt, nx),
        out_type=(
            jax.ShapeDtypeStruct((NC, N, D), jnp.float32),
            jax.ShapeDtypeStruct((NC, N, H), jnp.float32),
        ),
        mesh=mesh,
        scratch_types=[
            pltpu.VMEM_SHARED((N, D), jnp.float32),
            pltpu.VMEM_SHARED((N, H), jnp.float32),
            pltpu.VMEM((rpt, 128), jnp.int32),
            pltpu.VMEM((SK * 128, D), jnp.float32),
            pltpu.VMEM((SK * 128, H), jnp.float32),
        ],
        compiler_params=pltpu.CompilerParams(use_tc_tiling_on_sc=False),
    )
    return f(contrib, w, dst2, zacc, zs)


NB = 10


def _tc_combine(accs, ss, sel):
    k = len(accs)

    def body(*refs):
        a_refs = refs[:k]
        s_refs = refs[k:2 * k]
        sel_ref = refs[2 * k]
        out_ref = refs[2 * k + 1]
        acc = sum(r[0] + r[1] for r in a_refs[1:]) + a_refs[0][0] + \
            a_refs[0][1]
        sden = sum(r[0] + r[1] for r in s_refs[1:]) + s_refs[0][0] + \
            s_refs[0][1]
        swide = jnp.dot(sden, sel_ref[...],
                        preferred_element_type=jnp.float32)
        out_ref[...] = acc / (swide + 1e-16)

    bn = N // NB
    a_spec = pl.BlockSpec((NC, bn, D), lambda i: (0, i, 0))
    s_spec = pl.BlockSpec((NC, bn, H), lambda i: (0, i, 0))
    full = lambda shape: pl.BlockSpec(shape, lambda i: tuple(0 for _ in shape))
    return pl.pallas_call(
        body,
        grid=(NB,),
        in_specs=[a_spec] * k + [s_spec] * k + [full((H, D))],
        out_specs=pl.BlockSpec((bn, D), lambda i: (i, 0)),
        out_shape=jax.ShapeDtypeStruct((N, D), jnp.float32),
    )(*accs, *ss, sel)


def _node_body(nh0_ref, x_ref, sd_ref,
               c0_ref, c1_ref, f1w_ref, f1b_ref, f2w_ref, f2b_ref,
               g1_ref, b1_ref, g2_ref, b2_ref, out_ref):
    nh = nh0_ref[...]
    x = x_ref[...]
    nh = nh * (c0_ref[...] + sd_ref[...] * c1_ref[...])
    nh = nh + x
    h_res = nh

    m = jnp.mean(nh, axis=0, keepdims=True)
    v = jnp.mean((nh - m) * (nh - m), axis=0, keepdims=True)
    nh = (nh - m) / jnp.sqrt(v + 1e-5) * g1_ref[...] + b1_ref[...]

    nh = jnp.dot(nh, f1w_ref[...], preferred_element_type=jnp.float32)
    nh = jnp.maximum(nh + f1b_ref[...], 0.0)
    nh = jnp.dot(nh, f2w_ref[...], preferred_element_type=jnp.float32)
    nh = nh + f2b_ref[...] + h_res

    m2 = jnp.mean(nh, axis=0, keepdims=True)
    v2 = jnp.mean((nh - m2) * (nh - m2), axis=0, keepdims=True)
    out_ref[...] = (nh - m2) / jnp.sqrt(v2 + 1e-5) * g2_ref[...] + b2_ref[...]


def _tc_node(nh0, x, sqrt_deg, c0, c1, f1w, f1b, f2w, f2b,
             g1, b1, g2, b2):
    return pl.pallas_call(
        _node_body,
        out_shape=jax.ShapeDtypeStruct((N, D), jnp.float32),
        compiler_params=pltpu.CompilerParams(
            vmem_limit_bytes=60 * 1024 * 1024,
        ),
    )(nh0, x, sqrt_deg, c0, c1, f1w, f1b, f2w, f2b, g1, b1, g2, b2)


def kernel(x, full_index, full_conn, sqrt_deg, qkv_w, qkv_b, conn_w, conn_b,
           Aw, Bw, deg_coef, ffn1_w, ffn1_b, ffn2_w, ffn2_b, bn1_g, bn1_b,
           bn2_g, bn2_b):
    wqkv = qkv_w.T
    wc = conn_w.T
    bc = conn_b.reshape(1, 2 * D)
    hh = jnp.arange(D) // DH
    blk = (hh[:, None] == hh[None, :]).astype(jnp.float32)
    w2 = blk * jnp.tile(Bw.transpose(1, 0, 2).reshape(H * DH, DH), (1, H))
    a2 = (jnp.arange(H)[None, :] == hh[:, None]).astype(jnp.float32) * \
        Aw[:, :, 0].T.reshape(D, 1)
    sel = (hh[None, :] == jnp.arange(H)[:, None]).astype(jnp.float32)
    c0 = deg_coef[0, :, 0].reshape(1, D)
    c1 = deg_coef[0, :, 1].reshape(1, D)

    dst2 = full_index[0].reshape(NROW, 128)
    src2 = full_index[1].reshape(NROW, 128)
    zacc = jnp.zeros((N, D), jnp.float32)
    zs = jnp.zeros((N, H), jnp.float32)

    qh, kh, vh = _qkv(x, wqkv)

    gathered = [_sc_gather(qh, kh, vh, dst2, src2, row0, rpt, nx)
                for row0, rpt, nx in SLABS]
    conn2 = None
    accs, ss = [], []
    edged = []
    for (row0, rpt, nx), (qd, ks, vs) in zip(SLABS, gathered):
        conn2, w, contrib = _tc_edge(full_conn, qd, ks, vs, wc, bc, w2,
                                     a2, sel, row0 * 128 // EB, donor=conn2)
        edged.append((row0, rpt, nx, w, contrib))
    for row0, rpt, nx, w, contrib in edged:
        acc, s = _sc_scatter(contrib, w, dst2, zacc, zs, row0, rpt, nx)
        accs.append(acc)
        ss.append(s)
    nh0 = _tc_combine(accs, ss, sel)
    nh = _tc_node(nh0, x, sqrt_deg, c0, c1,
                  ffn1_w.T, ffn1_b.reshape(1, 2 * D),
                  ffn2_w.T, ffn2_b.reshape(1, D),
                  bn1_g.reshape(1, D), bn1_b.reshape(1, D),
                  bn2_g.reshape(1, D), bn2_b.reshape(1, D))
    return nh, conn2
